# Initial kernel scaffold; baseline (speedup 1.0000x reference)
import jax
import jax.numpy as jnp
from jax.experimental import pallas as pl


def kernel(x, edge_index, W, b):
    raise NotImplementedError("write your pallas kernel here")


# speedup vs baseline: 4.0371x; 4.0371x over previous
import functools

import jax
import jax.numpy as jnp
from jax import lax
from jax.experimental import pallas as pl
from jax.experimental.pallas import tpu as pltpu
from jax.experimental.pallas import tpu_sc as plsc

N = 10000
D = 128
E = 320000
K = 10
NC = 2
NS = 16
NW = NC * NS
NP = 10240
RPT = NP // NS
C = 128
NCH = 80
EP = NW * NCH * C

BR = 512
GR = NP // BR


def _edge_body(g_hbm, row_hbm, col_hbm, acc_hbm,
               row_v, col_v, gbuf, zbuf, acc_sh, gsem):
    cid = lax.axis_index("c")
    sid = lax.axis_index("s")

    zv = jnp.zeros((16,), jnp.float32)
    for i in range(16):
        for j in range(D // 16):
            zbuf[i, pl.ds(j * 16, 16)] = zv
    base = sid * RPT

    def _zero(i, carry):
        pltpu.sync_copy(zbuf, acc_sh.at[pl.ds(base + i * 16, 16)])
        return carry

    lax.fori_loop(0, RPT // 16, _zero, 0)

    pltpu.sync_copy(row_hbm.at[cid, sid], row_v)
    pltpu.sync_copy(col_hbm.at[cid, sid], col_v)
    plsc.subcore_barrier()

    def _chunk(j, carry):
        pltpu.async_copy(g_hbm.at[row_v.at[j]], gbuf, gsem).wait()
        pltpu.sync_copy(gbuf, acc_sh.at[col_v.at[j]], add=True)
        return carry

    lax.fori_loop(0, NCH, _chunk, 0)
    plsc.subcore_barrier()

    pltpu.sync_copy(acc_sh.at[pl.ds(base, RPT)],
                    acc_hbm.at[cid, pl.ds(base, RPT)])


_edge = functools.partial(
    pl.kernel,
    mesh=plsc.VectorSubcoreMesh(core_axis_name="c", subcore_axis_name="s"),
    out_type=jax.ShapeDtypeStruct((NC, NP, D), jnp.float32),
    scratch_types=[
        pltpu.VMEM((NCH, C), jnp.int32),
        pltpu.VMEM((NCH, C), jnp.int32),
        pltpu.VMEM((C, D), jnp.float32),
        pltpu.VMEM((16, D), jnp.float32),
        pltpu.VMEM_SHARED((NP, D), jnp.float32),
        pltpu.SemaphoreType.DMA,
    ],
)(_edge_body)


def _init_body(acc_ref, x_ref, wb_ref, dinv_ref, g_ref, oa_ref):
    i = pl.program_id(0)
    deg = acc_ref[0, :, 0:1] + acc_ref[1, :, 0:1] + 1.0
    rows = i * BR + lax.broadcasted_iota(jnp.int32, (BR, 1), 0)
    dinv = jnp.where(rows < N, lax.rsqrt(jnp.maximum(deg, 1e-12)), 0.0)
    x = x_ref[...]
    dinv_ref[...] = jnp.broadcast_to(dinv, (BR, D))
    g_ref[...] = dinv * x
    z = jnp.sum(x * wb_ref[0:1, :], axis=1, keepdims=True) + wb_ref[1, 0]
    oa_ref[...] = jax.nn.sigmoid(z) * x


def _comb_body(acc_ref, g_ref, dinv_ref, oa_ref, wb_ref, g_out_ref, oa_out_ref):
    t = acc_ref[0] + acc_ref[1] + g_ref[...]
    dinv = dinv_ref[...]
    h = dinv * t
    z = jnp.sum(h * wb_ref[0:1, :], axis=1, keepdims=True) + wb_ref[1, 0]
    oa_out_ref[...] = oa_ref[...] + jax.nn.sigmoid(z) * h
    g_out_ref[...] = dinv * h


def _init(acc, x_pad, wb):
    return pl.pallas_call(
        _init_body,
        grid=(GR,),
        in_specs=[
            pl.BlockSpec((NC, BR, D), lambda i: (0, i, 0)),
            pl.BlockSpec((BR, D), lambda i: (i, 0)),
            pl.BlockSpec((2, D), lambda i: (0, 0)),
        ],
        out_specs=[
            pl.BlockSpec((BR, D), lambda i: (i, 0)),
            pl.BlockSpec((BR, D), lambda i: (i, 0)),
            pl.BlockSpec((BR, D), lambda i: (i, 0)),
        ],
        out_shape=[jax.ShapeDtypeStruct((NP, D), jnp.float32)] * 3,
    )(acc, x_pad, wb)


def _comb(acc, g, dinv, oa, wb):
    return pl.pallas_call(
        _comb_body,
        grid=(GR,),
        in_specs=[
            pl.BlockSpec((NC, BR, D), lambda i: (0, i, 0)),
            pl.BlockSpec((BR, D), lambda i: (i, 0)),
            pl.BlockSpec((BR, D), lambda i: (i, 0)),
            pl.BlockSpec((BR, D), lambda i: (i, 0)),
            pl.BlockSpec((2, D), lambda i: (0, 0)),
        ],
        out_specs=[
            pl.BlockSpec((BR, D), lambda i: (i, 0)),
            pl.BlockSpec((BR, D), lambda i: (i, 0)),
        ],
        out_shape=[jax.ShapeDtypeStruct((NP, D), jnp.float32)] * 2,
    )(acc, g, dinv, oa, wb)


def kernel(x, edge_index, W, b):
    x = x.astype(jnp.float32)
    row = edge_index[0].astype(jnp.int32)
    col = edge_index[1].astype(jnp.int32)
    fill = jnp.full((EP - E,), N, jnp.int32)
    row_p = jnp.concatenate([row, fill]).reshape(NC, NS, NCH, C)
    col_p = jnp.concatenate([col, fill]).reshape(NC, NS, NCH, C)

    ids = lax.broadcasted_iota(jnp.int32, (NP, 1), 0)
    ones_t = jnp.broadcast_to(jnp.where(ids < N, 1.0, 0.0), (NP, D))
    x_pad = jnp.pad(x, ((0, NP - N), (0, 0)))
    wb = jnp.concatenate(
        [W.reshape(1, D), jnp.broadcast_to(b.reshape(1, 1), (1, D))], axis=0)

    acc = _edge(ones_t, row_p, col_p)
    dinv, g, oa = _init(acc, x_pad, wb)
    for _ in range(K):
        acc = _edge(g, row_p, col_p)
        g, oa = _comb(acc, g, dinv, oa, wb)
    return oa[:N]
